# baseline (speedup 1.0000x reference)
import functools

import jax
import jax.numpy as jnp
from jax import lax
from jax.experimental import pallas as pl
from jax.experimental.pallas import tpu as pltpu
from jax.experimental.pallas import tpu_sc as plsc

D_IN = 4096
D_OUT = 4096
NUMEL = D_OUT * D_IN

NC = 2
NS = 16
LANES = 16

REG_W = 1 << 20
NREG = NUMEL // REG_W
NPASS = NREG // NC
SLICE = REG_W // NS
DUMP = 2048
SPM_W = REG_W + 2 * DUMP
HOP_W = 1 << 14
HOPS = SLICE // HOP_W
WIN = 4928


def _make_sc_fused(NWIN):
    NV = WIN // LANES
    mesh = plsc.VectorSubcoreMesh(
        core_axis_name="c", subcore_axis_name="s", num_cores=NC, num_subcores=NS
    )

    @functools.partial(
        pl.kernel,
        mesh=mesh,
        out_type=jax.ShapeDtypeStruct((NUMEL,), jnp.float32),
        scratch_types=[
            pltpu.VMEM((WIN,), jnp.int32),
            pltpu.VMEM((WIN,), jnp.int32),
            pltpu.VMEM((WIN,), jnp.int32),
            pltpu.VMEM((WIN,), jnp.float32),
            pltpu.VMEM((WIN,), jnp.float32),
            pltpu.VMEM((WIN,), jnp.float32),
            pltpu.VMEM((HOP_W,), jnp.float32),
            pltpu.SemaphoreType.DMA,
            pltpu.SemaphoreType.DMA,
            pltpu.VMEM_SHARED((SPM_W,), jnp.float32),
        ],
    )
    def sc_fused(w_hbm, idx_hbm, vals_hbm, wmod_hbm, idx0, idx1, idx2,
                 val0, val1, val2, hop, sem_w, sem_s, spmem):
        cid = lax.axis_index("c")
        sid = lax.axis_index("s")
        ibufs = (idx0, idx1, idx2)
        vbufs = (val0, val1, val2)

        def each_buf(wb, fn):
            for k in range(3):
                @pl.when(wb == k)
                def _(k=k):
                    fn(ibufs[k], vbufs[k])

        def pass_body(p, c0):
            rbase = (p * NC + cid) * REG_W
            tbase = rbase + sid * SLICE
            sbase = DUMP + sid * SLICE

            def hop_in(h, c3):
                pltpu.sync_copy(w_hbm.at[pl.ds(tbase + h * HOP_W, HOP_W)],
                                hop)
                pltpu.sync_copy(hop,
                                spmem.at[pl.ds(sbase + h * HOP_W, HOP_W)])
                return c3

            lax.fori_loop(0, HOPS, hop_in, 0)
            plsc.subcore_barrier()

            pltpu.async_copy(idx_hbm.at[sid, 0], idx0, sem_w)
            pltpu.async_copy(vals_hbm.at[sid, 0], val0, sem_w)

            def win_body(w, c1):
                wb = lax.rem(w, 3)

                def wait_win(ib, vb):
                    pltpu.make_async_copy(idx_hbm.at[sid, 0], ib,
                                          sem_w).wait()
                    pltpu.make_async_copy(vals_hbm.at[sid, 0], vb,
                                          sem_w).wait()

                each_buf(wb, wait_win)

                @pl.when(w >= 2)
                def _():
                    pltpu.make_async_copy(val0, spmem.at[idx0], sem_s).wait()

                @pl.when(w + 1 < NWIN)
                def _():
                    def start_win(ib, vb):
                        pltpu.async_copy(idx_hbm.at[sid, w + 1], ib, sem_w)
                        pltpu.async_copy(vals_hbm.at[sid, w + 1], vb, sem_w)

                    each_buf(lax.rem(w + 1, 3), start_win)

                def compute_scatter(ib, vb):
                    def vbody(i, c2):
                        iv = ib[pl.ds(i * LANES, LANES)]
                        loc = iv - (rbase - DUMP)
                        spread = iv & jnp.int32(DUMP - 1)
                        loc = lax.max(loc, spread)
                        loc = lax.min(loc, spread + jnp.int32(REG_W + DUMP))
                        ib[pl.ds(i * LANES, LANES)] = loc
                        return c2

                    lax.fori_loop(0, NV, vbody, 0)
                    pltpu.async_copy(vb, spmem.at[ib], sem_s, add=True)

                each_buf(wb, compute_scatter)
                return c1

            lax.fori_loop(0, NWIN, win_body, 0)
            pltpu.make_async_copy(val0, spmem.at[idx0], sem_s).wait()
            pltpu.make_async_copy(val0, spmem.at[idx0], sem_s).wait()
            plsc.subcore_barrier()

            def hop_out(h, c4):
                pltpu.sync_copy(spmem.at[pl.ds(sbase + h * HOP_W, HOP_W)],
                                hop)
                pltpu.sync_copy(hop,
                                wmod_hbm.at[pl.ds(tbase + h * HOP_W, HOP_W)])
                return c4

            lax.fori_loop(0, HOPS, hop_out, 0)
            plsc.subcore_barrier()
            return c0

        lax.fori_loop(0, NPASS, pass_body, 0)

    return sc_fused


def _mm_body(x_ref, w_ref, b_ref, o_ref):
    acc = lax.dot_general(
        x_ref[...],
        w_ref[...],
        dimension_numbers=(((1,), (1,)), ((), ())),
        preferred_element_type=jnp.float32,
    )
    o_ref[...] = acc + b_ref[...][None, :]


def _tc_matmul(x, w2d, b):
    bn = 512
    batch = x.shape[0]
    return pl.pallas_call(
        _mm_body,
        grid=(D_OUT // bn,),
        in_specs=[
            pl.BlockSpec((batch, D_IN), lambda i: (0, 0)),
            pl.BlockSpec((bn, D_IN), lambda i: (i, 0)),
            pl.BlockSpec((bn,), lambda i: (i,)),
        ],
        out_specs=pl.BlockSpec((batch, bn), lambda i: (0, i)),
        out_shape=jax.ShapeDtypeStruct((batch, D_OUT), jnp.float32),
    )(x, w2d, b)


def kernel(x, W_flat, b, flip_vals, flip_idx):
    n = flip_idx.shape[0]
    chunk_q = NS * WIN
    nwin = -(-n // chunk_q)
    CH = nwin * WIN
    npad = CH * NS - n

    idx = flip_idx.astype(jnp.int32)
    vals = flip_vals.astype(jnp.float32)
    if npad:
        idx = jnp.concatenate([idx, jnp.broadcast_to(idx[0], (npad,))])
        vals = jnp.concatenate([vals, jnp.zeros((npad,), jnp.float32)])
    idx3 = idx.reshape(NS, nwin, WIN)
    vals3 = vals.reshape(NS, nwin, WIN)

    w_mod = _make_sc_fused(nwin)(W_flat, idx3, vals3)
    return _tc_matmul(x, w_mod.reshape(D_OUT, D_IN), b)


# speedup vs baseline: 1.1578x; 1.1578x over previous
import functools

import jax
import jax.numpy as jnp
from jax import lax
from jax.experimental import pallas as pl
from jax.experimental.pallas import tpu as pltpu
from jax.experimental.pallas import tpu_sc as plsc

D_IN = 4096
D_OUT = 4096
NUMEL = D_OUT * D_IN

NC = 2
NS = 16
LANES = 16

REG_W = 1 << 20
NREG = NUMEL // REG_W
NPASS = NREG // NC
SLICE = REG_W // NS
DUMP = 2048
SPM_W = REG_W + 2 * DUMP
HOP_W = 1 << 14
HOPS = SLICE // HOP_W
HPAIR = HOPS // 2
WIN = 8064


def _make_sc_fused(NWIN):
    NV = WIN // LANES
    NPAIR = NWIN // 2
    mesh = plsc.VectorSubcoreMesh(
        core_axis_name="c", subcore_axis_name="s", num_cores=NC, num_subcores=NS
    )

    @functools.partial(
        pl.kernel,
        mesh=mesh,
        out_type=jax.ShapeDtypeStruct((NUMEL,), jnp.float32),
        scratch_types=[
            pltpu.VMEM((WIN,), jnp.int32),
            pltpu.VMEM((WIN,), jnp.int32),
            pltpu.VMEM((WIN,), jnp.float32),
            pltpu.VMEM((WIN,), jnp.float32),
            pltpu.VMEM((HOP_W,), jnp.float32),
            pltpu.VMEM((HOP_W,), jnp.float32),
            pltpu.SemaphoreType.DMA,
            pltpu.SemaphoreType.DMA,
            pltpu.SemaphoreType.DMA,
            pltpu.SemaphoreType.DMA,
            pltpu.SemaphoreType.DMA,
            pltpu.SemaphoreType.DMA,
            pltpu.SemaphoreType.DMA,
            pltpu.SemaphoreType.DMA,
            pltpu.VMEM_SHARED((SPM_W,), jnp.float32),
        ],
    )
    def sc_fused(w_hbm, idx_hbm, vals_hbm, wmod_hbm, idxA, idxB, valA, valB,
                 hopA, hopB, s_da, s_db, s_sa, s_sb, s_1a, s_1b, s_2a, s_2b,
                 spmem):
        cid = lax.axis_index("c")
        sid = lax.axis_index("s")

        def stage(src, dst):
            pltpu.async_copy(src.at[pl.ds(0, HOP_W)], hopA, s_1a)

            def hbody(hp, c):
                h0 = hp * 2
                pltpu.make_async_copy(
                    src.at[pl.ds(0, HOP_W)], hopA, s_1a).wait()

                @pl.when(hp > 0)
                def _():
                    pltpu.make_async_copy(
                        hopB, dst.at[pl.ds(0, HOP_W)], s_2b).wait()

                pltpu.async_copy(
                    src.at[pl.ds((h0 + 1) * HOP_W, HOP_W)], hopB, s_1b)
                pltpu.async_copy(
                    hopA, dst.at[pl.ds(h0 * HOP_W, HOP_W)], s_2a)
                pltpu.make_async_copy(
                    src.at[pl.ds(0, HOP_W)], hopB, s_1b).wait()
                pltpu.async_copy(
                    hopB, dst.at[pl.ds((h0 + 1) * HOP_W, HOP_W)], s_2b)
                pltpu.make_async_copy(
                    hopA, dst.at[pl.ds(0, HOP_W)], s_2a).wait()

                @pl.when(hp + 1 < HPAIR)
                def _():
                    pltpu.async_copy(
                        src.at[pl.ds((h0 + 2) * HOP_W, HOP_W)], hopA, s_1a)

                return c

            lax.fori_loop(0, HPAIR, hbody, 0)
            pltpu.make_async_copy(hopB, dst.at[pl.ds(0, HOP_W)], s_2b).wait()

        def clamp_compute(ib, rbase):
            def vbody(i, c2):
                iv = ib[pl.ds(i * LANES, LANES)]
                loc = iv - (rbase - DUMP)
                spread = iv & jnp.int32(DUMP - 1)
                loc = lax.max(loc, spread)
                loc = lax.min(loc, spread + jnp.int32(REG_W + DUMP))
                ib[pl.ds(i * LANES, LANES)] = loc
                return c2

            lax.fori_loop(0, NV, vbody, 0)

        def pass_body(p, c0):
            rbase = (p * NC + cid) * REG_W
            tbase = rbase + sid * SLICE
            sbase = DUMP + sid * SLICE

            stage(w_hbm.at[pl.ds(tbase, SLICE)], spmem.at[pl.ds(sbase, SLICE)])
            plsc.subcore_barrier()

            pltpu.async_copy(idx_hbm.at[sid, 0], idxA, s_da)
            pltpu.async_copy(vals_hbm.at[sid, 0], valA, s_da)

            def win_pair(wp, c1):
                w0 = wp * 2
                pltpu.make_async_copy(idx_hbm.at[sid, 0], idxA, s_da).wait()
                pltpu.make_async_copy(vals_hbm.at[sid, 0], valA, s_da).wait()

                @pl.when(wp > 0)
                def _():
                    pltpu.make_async_copy(valB, spmem.at[idxB], s_sb).wait()

                pltpu.async_copy(idx_hbm.at[sid, w0 + 1], idxB, s_db)
                pltpu.async_copy(vals_hbm.at[sid, w0 + 1], valB, s_db)
                clamp_compute(idxA, rbase)
                pltpu.async_copy(valA, spmem.at[idxA], s_sa, add=True)
                pltpu.make_async_copy(idx_hbm.at[sid, 0], idxB, s_db).wait()
                pltpu.make_async_copy(vals_hbm.at[sid, 0], valB, s_db).wait()
                clamp_compute(idxB, rbase)
                pltpu.async_copy(valB, spmem.at[idxB], s_sb, add=True)
                pltpu.make_async_copy(valA, spmem.at[idxA], s_sa).wait()

                @pl.when(wp + 1 < NPAIR)
                def _():
                    pltpu.async_copy(idx_hbm.at[sid, w0 + 2], idxA, s_da)
                    pltpu.async_copy(vals_hbm.at[sid, w0 + 2], valA, s_da)

                return c1

            lax.fori_loop(0, NPAIR, win_pair, 0)
            pltpu.make_async_copy(valB, spmem.at[idxB], s_sb).wait()
            plsc.subcore_barrier()

            stage(spmem.at[pl.ds(sbase, SLICE)],
                  wmod_hbm.at[pl.ds(tbase, SLICE)])
            plsc.subcore_barrier()
            return c0

        lax.fori_loop(0, NPASS, pass_body, 0)

    return sc_fused


def _mm_body(x_ref, w_ref, b_ref, o_ref):
    acc = lax.dot_general(
        x_ref[...],
        w_ref[...],
        dimension_numbers=(((1,), (1,)), ((), ())),
        preferred_element_type=jnp.float32,
    )
    o_ref[...] = acc + b_ref[...][None, :]


def _tc_matmul(x, w2d, b):
    bn = 512
    batch = x.shape[0]
    return pl.pallas_call(
        _mm_body,
        grid=(D_OUT // bn,),
        in_specs=[
            pl.BlockSpec((batch, D_IN), lambda i: (0, 0)),
            pl.BlockSpec((bn, D_IN), lambda i: (i, 0)),
            pl.BlockSpec((bn,), lambda i: (i,)),
        ],
        out_specs=pl.BlockSpec((batch, bn), lambda i: (0, i)),
        out_shape=jax.ShapeDtypeStruct((batch, D_OUT), jnp.float32),
    )(x, w2d, b)


def kernel(x, W_flat, b, flip_vals, flip_idx):
    n = flip_idx.shape[0]
    chunk_q = NS * WIN
    nwin = -(-n // chunk_q)
    if nwin % 2:
        nwin += 1
    CH = nwin * WIN
    npad = CH * NS - n

    idx = flip_idx.astype(jnp.int32)
    vals = flip_vals.astype(jnp.float32)
    if npad:
        idx = jnp.concatenate([idx, jnp.broadcast_to(idx[0], (npad,))])
        vals = jnp.concatenate([vals, jnp.zeros((npad,), jnp.float32)])
    idx3 = idx.reshape(NS, nwin, WIN)
    vals3 = vals.reshape(NS, nwin, WIN)

    w_mod = _make_sc_fused(nwin)(W_flat, idx3, vals3)
    return _tc_matmul(x, w_mod.reshape(D_OUT, D_IN), b)


# speedup vs baseline: 1.3666x; 1.1804x over previous
import functools

import jax
import jax.numpy as jnp
from jax import lax
from jax.experimental import pallas as pl
from jax.experimental.pallas import tpu as pltpu
from jax.experimental.pallas import tpu_sc as plsc

D_IN = 4096
D_OUT = 4096
NUMEL = D_OUT * D_IN

NC = 2
NS = 16
LANES = 16

REG_W = 1 << 20
NREG = NUMEL // REG_W
NPASS = NREG // NC
SLICE = REG_W // NS
DUMP = 2048
SPM_W = REG_W + 2 * DUMP
HOP_W = 1 << 14
HOPS = SLICE // HOP_W
WIN = 11808


def _make_sc_fused(NWIN):
    NV = WIN // LANES
    mesh = plsc.VectorSubcoreMesh(
        core_axis_name="c", subcore_axis_name="s", num_cores=NC, num_subcores=NS
    )

    @functools.partial(
        pl.kernel,
        mesh=mesh,
        out_type=jax.ShapeDtypeStruct((NUMEL,), jnp.float32),
        scratch_types=[
            pltpu.VMEM((WIN,), jnp.int32),
            pltpu.VMEM((WIN,), jnp.float32),
            pltpu.VMEM((WIN,), jnp.int32),
            pltpu.VMEM((HOP_W,), jnp.float32),
            pltpu.VMEM_SHARED((SPM_W,), jnp.float32),
        ],
    )
    def sc_fused(w_hbm, idx_hbm, vals_hbm, wmod_hbm, idxw, valsw, ibuf,
                 hop, spmem):
        cid = lax.axis_index("c")
        sid = lax.axis_index("s")

        def pass_body(p, c0):
            rbase = (p * NC + cid) * REG_W
            tbase = rbase + sid * SLICE
            sbase = DUMP + sid * SLICE

            def hop_in(h, c3):
                pltpu.sync_copy(w_hbm.at[pl.ds(tbase + h * HOP_W, HOP_W)],
                                hop)
                pltpu.sync_copy(hop,
                                spmem.at[pl.ds(sbase + h * HOP_W, HOP_W)])
                return c3

            lax.fori_loop(0, HOPS, hop_in, 0)
            plsc.subcore_barrier()

            def win_body(w, c1):
                pltpu.sync_copy(idx_hbm.at[sid, w], idxw)
                pltpu.sync_copy(vals_hbm.at[sid, w], valsw)

                def vbody(i, c2):
                    iv = idxw[pl.ds(i * LANES, LANES)]
                    loc = iv - (rbase - DUMP)
                    spread = iv & jnp.int32(DUMP - 1)
                    loc = lax.max(loc, spread)
                    loc = lax.min(loc, spread + jnp.int32(REG_W + DUMP))
                    ibuf[pl.ds(i * LANES, LANES)] = loc
                    return c2

                lax.fori_loop(0, NV, vbody, 0)
                pltpu.sync_copy(valsw, spmem.at[ibuf], add=True)
                return c1

            lax.fori_loop(0, NWIN, win_body, 0)
            plsc.subcore_barrier()

            def hop_out(h, c4):
                pltpu.sync_copy(spmem.at[pl.ds(sbase + h * HOP_W, HOP_W)],
                                hop)
                pltpu.sync_copy(hop,
                                wmod_hbm.at[pl.ds(tbase + h * HOP_W, HOP_W)])
                return c4

            lax.fori_loop(0, HOPS, hop_out, 0)
            plsc.subcore_barrier()
            return c0

        lax.fori_loop(0, NPASS, pass_body, 0)

    return sc_fused


def _mm_body(x_ref, w_ref, b_ref, o_ref):
    acc = lax.dot_general(
        x_ref[...],
        w_ref[...],
        dimension_numbers=(((1,), (1,)), ((), ())),
        preferred_element_type=jnp.float32,
    )
    o_ref[...] = acc + b_ref[...][None, :]


def _tc_matmul(x, w2d, b):
    bn = 512
    batch = x.shape[0]
    return pl.pallas_call(
        _mm_body,
        grid=(D_OUT // bn,),
        in_specs=[
            pl.BlockSpec((batch, D_IN), lambda i: (0, 0)),
            pl.BlockSpec((bn, D_IN), lambda i: (i, 0)),
            pl.BlockSpec((bn,), lambda i: (i,)),
        ],
        out_specs=pl.BlockSpec((batch, bn), lambda i: (0, i)),
        out_shape=jax.ShapeDtypeStruct((batch, D_OUT), jnp.float32),
    )(x, w2d, b)


def kernel(x, W_flat, b, flip_vals, flip_idx):
    n = flip_idx.shape[0]
    chunk_q = NS * WIN
    nwin = -(-n // chunk_q)
    CH = nwin * WIN
    npad = CH * NS - n

    idx = flip_idx.astype(jnp.int32)
    vals = flip_vals.astype(jnp.float32)
    if npad:
        idx = jnp.concatenate([idx, jnp.broadcast_to(idx[0], (npad,))])
        vals = jnp.concatenate([vals, jnp.zeros((npad,), jnp.float32)])
    idx3 = idx.reshape(NS, nwin, WIN)
    vals3 = vals.reshape(NS, nwin, WIN)

    w_mod = _make_sc_fused(nwin)(W_flat, idx3, vals3)
    return _tc_matmul(x, w_mod.reshape(D_OUT, D_IN), b)


# speedup vs baseline: 1.4022x; 1.0260x over previous
import functools

import jax
import jax.numpy as jnp
from jax import lax
from jax.experimental import pallas as pl
from jax.experimental.pallas import tpu as pltpu
from jax.experimental.pallas import tpu_sc as plsc

D_IN = 4096
D_OUT = 4096
NUMEL = D_OUT * D_IN

NC = 2
NS = 16
LANES = 16

REG_W = 1 << 20
NREG = NUMEL // REG_W
NPASS = NREG // NC
SLICE = REG_W // NS
DUMP = 2048
SPM_W = REG_W + 2 * DUMP
HOP_W = 1 << 14
HOPS = SLICE // HOP_W
WIN = 15744


def _make_sc_fused(NWIN):
    NV = WIN // LANES
    mesh = plsc.VectorSubcoreMesh(
        core_axis_name="c", subcore_axis_name="s", num_cores=NC, num_subcores=NS
    )

    @functools.partial(
        pl.kernel,
        mesh=mesh,
        out_type=jax.ShapeDtypeStruct((NUMEL,), jnp.float32),
        scratch_types=[
            pltpu.VMEM((WIN,), jnp.int32),
            pltpu.VMEM((WIN,), jnp.float32),
            pltpu.VMEM((WIN,), jnp.int32),
            pltpu.VMEM((HOP_W,), jnp.float32),
            pltpu.VMEM_SHARED((SPM_W,), jnp.float32),
        ],
    )
    def sc_fused(w_hbm, idx_hbm, vals_hbm, wmod_hbm, idxw, valsw, ibuf,
                 hop, spmem):
        cid = lax.axis_index("c")
        sid = lax.axis_index("s")

        def pass_body(p, c0):
            rbase = (p * NC + cid) * REG_W
            tbase = rbase + sid * SLICE
            sbase = DUMP + sid * SLICE

            def hop_in(h, c3):
                pltpu.sync_copy(w_hbm.at[pl.ds(tbase + h * HOP_W, HOP_W)],
                                hop)
                pltpu.sync_copy(hop,
                                spmem.at[pl.ds(sbase + h * HOP_W, HOP_W)])
                return c3

            lax.fori_loop(0, HOPS, hop_in, 0)
            plsc.subcore_barrier()

            def win_body(w, c1):
                pltpu.sync_copy(idx_hbm.at[sid, w], idxw)
                pltpu.sync_copy(vals_hbm.at[sid, w], valsw)

                def vbody(i, c2):
                    iv = idxw[pl.ds(i * LANES, LANES)]
                    loc = iv - (rbase - DUMP)
                    spread = iv & jnp.int32(DUMP - 1)
                    loc = lax.max(loc, spread)
                    loc = lax.min(loc, spread + jnp.int32(REG_W + DUMP))
                    ibuf[pl.ds(i * LANES, LANES)] = loc
                    return c2

                lax.fori_loop(0, NV, vbody, 0)
                pltpu.sync_copy(valsw, spmem.at[ibuf], add=True)
                return c1

            lax.fori_loop(0, NWIN, win_body, 0)
            plsc.subcore_barrier()

            def hop_out(h, c4):
                pltpu.sync_copy(spmem.at[pl.ds(sbase + h * HOP_W, HOP_W)],
                                hop)
                pltpu.sync_copy(hop,
                                wmod_hbm.at[pl.ds(tbase + h * HOP_W, HOP_W)])
                return c4

            lax.fori_loop(0, HOPS, hop_out, 0)
            plsc.subcore_barrier()
            return c0

        lax.fori_loop(0, NPASS, pass_body, 0)

    return sc_fused


def _mm_body(x_ref, w_ref, b_ref, o_ref):
    acc = lax.dot_general(
        x_ref[...],
        w_ref[...],
        dimension_numbers=(((1,), (1,)), ((), ())),
        preferred_element_type=jnp.float32,
    )
    o_ref[...] = acc + b_ref[...][None, :]


def _tc_matmul(x, w2d, b):
    bn = 512
    batch = x.shape[0]
    return pl.pallas_call(
        _mm_body,
        grid=(D_OUT // bn,),
        in_specs=[
            pl.BlockSpec((batch, D_IN), lambda i: (0, 0)),
            pl.BlockSpec((bn, D_IN), lambda i: (i, 0)),
            pl.BlockSpec((bn,), lambda i: (i,)),
        ],
        out_specs=pl.BlockSpec((batch, bn), lambda i: (0, i)),
        out_shape=jax.ShapeDtypeStruct((batch, D_OUT), jnp.float32),
    )(x, w2d, b)


def kernel(x, W_flat, b, flip_vals, flip_idx):
    n = flip_idx.shape[0]
    chunk_q = NS * WIN
    nwin = -(-n // chunk_q)
    CH = nwin * WIN
    npad = CH * NS - n

    idx = flip_idx.astype(jnp.int32)
    vals = flip_vals.astype(jnp.float32)
    if npad:
        idx = jnp.concatenate([idx, jnp.broadcast_to(idx[0], (npad,))])
        vals = jnp.concatenate([vals, jnp.zeros((npad,), jnp.float32)])
    idx3 = idx.reshape(NS, nwin, WIN)
    vals3 = vals.reshape(NS, nwin, WIN)

    w_mod = _make_sc_fused(nwin)(W_flat, idx3, vals3)
    return _tc_matmul(x, w_mod.reshape(D_OUT, D_IN), b)


# speedup vs baseline: 1.5677x; 1.1180x over previous
import functools

import jax
import jax.numpy as jnp
from jax import lax
from jax.experimental import pallas as pl
from jax.experimental.pallas import tpu as pltpu
from jax.experimental.pallas import tpu_sc as plsc

D_IN = 4096
D_OUT = 4096
NUMEL = D_OUT * D_IN

NC = 2
NS = 16
LANES = 16

REG_W = 1 << 20
NREG = NUMEL // REG_W
NPASS = NREG // NC
SLICE = REG_W // NS
DUMP = 2048
SPM_W = REG_W + 2 * DUMP
HOP_W = 1 << 14
HOPS = SLICE // HOP_W
WIN = 15744


def _make_sc_fused(NWIN):
    NV = WIN // LANES
    mesh = plsc.VectorSubcoreMesh(
        core_axis_name="c", subcore_axis_name="s", num_cores=NC, num_subcores=NS
    )

    CHT = NWIN * WIN

    @functools.partial(
        pl.kernel,
        mesh=mesh,
        out_type=jax.ShapeDtypeStruct((NUMEL,), jnp.float32),
        scratch_types=[
            pltpu.VMEM((WIN,), jnp.int32),
            pltpu.VMEM((WIN,), jnp.float32),
            pltpu.VMEM((WIN,), jnp.int32),
            pltpu.VMEM((HOP_W,), jnp.float32),
            pltpu.VMEM_SHARED((SPM_W,), jnp.float32),
        ],
    )
    def sc_fused(w_hbm, idx_hbm, vals_hbm, wmod_hbm, idxw, valsw, ibuf,
                 hop, spmem):
        cid = lax.axis_index("c")
        sid = lax.axis_index("s")

        def pass_body(p, c0):
            rbase = (p * NC + cid) * REG_W
            tbase = rbase + sid * SLICE
            sbase = DUMP + sid * SLICE

            def hop_in(h, c3):
                pltpu.sync_copy(w_hbm.at[pl.ds(tbase + h * HOP_W, HOP_W)],
                                hop)
                pltpu.sync_copy(hop,
                                spmem.at[pl.ds(sbase + h * HOP_W, HOP_W)])
                return c3

            lax.fori_loop(0, HOPS, hop_in, 0)
            plsc.subcore_barrier()

            def win_body(w, c1):
                fo = sid * CHT + w * WIN
                pltpu.sync_copy(idx_hbm.at[pl.ds(fo, WIN)], idxw)
                pltpu.sync_copy(vals_hbm.at[pl.ds(fo, WIN)], valsw)

                def vbody(i, c2):
                    iv = idxw[pl.ds(i * LANES, LANES)]
                    loc = iv - (rbase - DUMP)
                    spread = iv & jnp.int32(DUMP - 1)
                    loc = lax.max(loc, spread)
                    loc = lax.min(loc, spread + jnp.int32(REG_W + DUMP))
                    ibuf[pl.ds(i * LANES, LANES)] = loc
                    return c2

                lax.fori_loop(0, NV, vbody, 0)
                pltpu.sync_copy(valsw, spmem.at[ibuf], add=True)
                return c1

            lax.fori_loop(0, NWIN, win_body, 0)
            plsc.subcore_barrier()

            def hop_out(h, c4):
                pltpu.sync_copy(spmem.at[pl.ds(sbase + h * HOP_W, HOP_W)],
                                hop)
                pltpu.sync_copy(hop,
                                wmod_hbm.at[pl.ds(tbase + h * HOP_W, HOP_W)])
                return c4

            lax.fori_loop(0, HOPS, hop_out, 0)
            plsc.subcore_barrier()
            return c0

        lax.fori_loop(0, NPASS, pass_body, 0)

    return sc_fused


def _mm_body(bn, x_ref, w_ref, b_ref, o_ref):
    w = w_ref[...].reshape(bn, D_IN)
    acc = lax.dot_general(
        x_ref[...],
        w,
        dimension_numbers=(((1,), (1,)), ((), ())),
        preferred_element_type=jnp.float32,
    )
    o_ref[...] = acc + b_ref[...][None, :]


def _tc_matmul(x, w_flat, b):
    bn = 512
    batch = x.shape[0]
    return pl.pallas_call(
        functools.partial(_mm_body, bn),
        grid=(D_OUT // bn,),
        in_specs=[
            pl.BlockSpec((batch, D_IN), lambda i: (0, 0)),
            pl.BlockSpec((bn * D_IN,), lambda i: (i,)),
            pl.BlockSpec((bn,), lambda i: (i,)),
        ],
        out_specs=pl.BlockSpec((batch, bn), lambda i: (0, i)),
        out_shape=jax.ShapeDtypeStruct((batch, D_OUT), jnp.float32),
    )(x, w_flat, b)


def kernel(x, W_flat, b, flip_vals, flip_idx):
    n = flip_idx.shape[0]
    chunk_q = NS * WIN
    nwin = -(-n // chunk_q)
    CH = nwin * WIN
    npad = CH * NS - n

    idx = flip_idx.astype(jnp.int32)
    vals = flip_vals.astype(jnp.float32)
    if npad:
        idx = jnp.concatenate([idx, jnp.broadcast_to(idx[0], (npad,))])
        vals = jnp.concatenate([vals, jnp.zeros((npad,), jnp.float32)])
    w_mod = _make_sc_fused(nwin)(W_flat, idx, vals)
    return _tc_matmul(x, w_mod, b)


# speedup vs baseline: 1.7188x; 1.0964x over previous
import functools

import jax
import jax.numpy as jnp
from jax import lax
from jax.experimental import pallas as pl
from jax.experimental.pallas import tpu as pltpu
from jax.experimental.pallas import tpu_sc as plsc

D_IN = 4096
D_OUT = 4096
NUMEL = D_OUT * D_IN

NC = 2
NS = 16
LANES = 16

HOP_W = 8192
UNIT = NS * HOP_W
HALF_U = NUMEL // NC // UNIT
PASS_U = 11
NPASS = -(-HALF_U // PASS_U)
REG_MAX = PASS_U * UNIT
DUMP = 2048
SPM_W = REG_MAX + 2 * DUMP
WIN = 7872


def _make_sc_fused(NWIN):
    NV = WIN // LANES
    mesh = plsc.VectorSubcoreMesh(
        core_axis_name="c", subcore_axis_name="s", num_cores=NC, num_subcores=NS
    )

    CHT = NWIN * WIN

    @functools.partial(
        pl.kernel,
        mesh=mesh,
        out_type=jax.ShapeDtypeStruct((NUMEL,), jnp.float32),
        scratch_types=[
            pltpu.VMEM((WIN,), jnp.int32),
            pltpu.VMEM((WIN,), jnp.float32),
            pltpu.VMEM((WIN,), jnp.int32),
            pltpu.VMEM((HOP_W,), jnp.float32),
            pltpu.VMEM_SHARED((SPM_W,), jnp.float32),
        ],
    )
    def sc_fused(w_hbm, idx_hbm, vals_hbm, wmod_hbm, idxw, valsw, ibuf,
                 hop, spmem):
        cid = lax.axis_index("c")
        sid = lax.axis_index("s")

        def pass_body(p, c0):
            su = jnp.minimum(jnp.int32(PASS_U),
                             jnp.int32(HALF_U) - p * PASS_U)
            rsize = su * jnp.int32(UNIT)
            rbase = cid * jnp.int32(NUMEL // NC) + p * jnp.int32(PASS_U * UNIT)
            tbase = rbase + sid * (su * jnp.int32(HOP_W))
            sbase = jnp.int32(DUMP) + sid * (su * jnp.int32(HOP_W))

            def hop_in(h, c3):
                pltpu.sync_copy(w_hbm.at[pl.ds(tbase + h * HOP_W, HOP_W)],
                                hop)
                pltpu.sync_copy(hop,
                                spmem.at[pl.ds(sbase + h * HOP_W, HOP_W)])
                return c3

            lax.fori_loop(0, su, hop_in, 0)
            plsc.subcore_barrier()

            def win_body(w, c1):
                fo = sid * CHT + w * WIN
                pltpu.sync_copy(idx_hbm.at[pl.ds(fo, WIN)], idxw)
                pltpu.sync_copy(vals_hbm.at[pl.ds(fo, WIN)], valsw)

                hi = spread_hi = rsize + jnp.int32(DUMP)

                def vbody(i, c2):
                    iv = idxw[pl.ds(i * LANES, LANES)]
                    loc = iv - (rbase - jnp.int32(DUMP))
                    spread = iv & jnp.int32(DUMP - 1)
                    loc = lax.max(loc, spread)
                    loc = lax.min(loc, spread + hi)
                    ibuf[pl.ds(i * LANES, LANES)] = loc
                    return c2

                lax.fori_loop(0, NV, vbody, 0)
                pltpu.sync_copy(valsw, spmem.at[ibuf], add=True)
                return c1

            lax.fori_loop(0, NWIN, win_body, 0)
            plsc.subcore_barrier()

            def hop_out(h, c4):
                pltpu.sync_copy(spmem.at[pl.ds(sbase + h * HOP_W, HOP_W)],
                                hop)
                pltpu.sync_copy(hop,
                                wmod_hbm.at[pl.ds(tbase + h * HOP_W, HOP_W)])
                return c4

            lax.fori_loop(0, su, hop_out, 0)
            plsc.subcore_barrier()
            return c0

        lax.fori_loop(0, NPASS, pass_body, 0)

    return sc_fused


def _mm_body(bn, x_ref, w_ref, b_ref, o_ref):
    w = w_ref[...].reshape(bn, D_IN)
    acc = lax.dot_general(
        x_ref[...],
        w,
        dimension_numbers=(((1,), (1,)), ((), ())),
        preferred_element_type=jnp.float32,
    )
    o_ref[...] = acc + b_ref[...][None, :]


def _tc_matmul(x, w_flat, b):
    bn = 512
    batch = x.shape[0]
    return pl.pallas_call(
        functools.partial(_mm_body, bn),
        grid=(D_OUT // bn,),
        in_specs=[
            pl.BlockSpec((batch, D_IN), lambda i: (0, 0)),
            pl.BlockSpec((bn * D_IN,), lambda i: (i,)),
            pl.BlockSpec((bn,), lambda i: (i,)),
        ],
        out_specs=pl.BlockSpec((batch, bn), lambda i: (0, i)),
        out_shape=jax.ShapeDtypeStruct((batch, D_OUT), jnp.float32),
    )(x, w_flat, b)


def kernel(x, W_flat, b, flip_vals, flip_idx):
    n = flip_idx.shape[0]
    chunk_q = NS * WIN
    nwin = -(-n // chunk_q)
    CH = nwin * WIN
    npad = CH * NS - n

    idx = flip_idx.astype(jnp.int32)
    vals = flip_vals.astype(jnp.float32)
    if npad:
        idx = jnp.concatenate([idx, jnp.broadcast_to(idx[0], (npad,))])
        vals = jnp.concatenate([vals, jnp.zeros((npad,), jnp.float32)])
    w_mod = _make_sc_fused(nwin)(W_flat, idx, vals)
    return _tc_matmul(x, w_mod, b)
